# baseline (speedup 1.0000x reference)
import functools

import jax
import jax.numpy as jnp
from jax import lax
from jax.experimental import pallas as pl
from jax.experimental.pallas import tpu as pltpu
from jax.experimental.pallas import tpu_sc as plsc

B, T, H, C = 8, 2048, 128, 32
ENC = 1024
TM = T - ENC
QM = TM // 128

NC, NS = 2, 16
NW = NC * NS
ROWS_PER_W = (B * C) // NW
QT = T // 128


def _sc_gather_body(shuffle_hbm, spikes_hbm, out_hbm, idx_v, plane_v, res_v, sem):
    wid = lax.axis_index("s") * NC + lax.axis_index("c")
    r0 = wid * ROWS_PER_W
    b = r0 // C
    c0 = r0 % C
    pltpu.sync_copy(shuffle_hbm.at[pl.ds(ENC, TM)], idx_v)
    cps = [pltpu.async_copy(
        spikes_hbm.at[b].at[c0 // 8].at[:, cl],
        plane_v.at[cl], sem) for cl in range(8)]
    for cp in cps:
        cp.wait()
    for g in range(TM // 16):
        t16 = idx_v[pl.ds(g * 16, 16)]
        q16 = lax.shift_right_logical(t16, 7)
        l16 = lax.bitwise_and(t16, 127)
        dst = ((g // 8) * 8, pl.ds((g % 8) * 16, 16))
        for cl in range(8):
            v16 = plsc.load_gather(plane_v, [jnp.full((16,), cl, jnp.int32), q16, l16])
            res_v[(dst[0] + cl, dst[1])] = v16
    for q in range(QM):
        pltpu.sync_copy(res_v.at[pl.ds(q * 8, 8)],
                        out_hbm.at[b].at[q].at[pl.ds(c0, 8)])


_sc_gather = functools.partial(
    pl.kernel,
    mesh=plsc.VectorSubcoreMesh(core_axis_name="c", subcore_axis_name="s"),
    out_type=jax.ShapeDtypeStruct((B, QM, C, 128), jnp.int32),
    scratch_types=[
        pltpu.VMEM((TM,), jnp.int32),
        pltpu.VMEM((8, QT, 128), jnp.int32),
        pltpu.VMEM((QM * 8, 128), jnp.int32),
        pltpu.SemaphoreType.DMA,
    ],
    compiler_params=pltpu.CompilerParams(use_tc_tiling_on_sc=False,
                                         needs_layout_passes=False),
)(_sc_gather_body)


def _tc_mlp_body(lengths_ref, tokpos_ref, bf_ref, w1_ref, b1_ref, w2_ref, b2_ref,
                 lr_ref, eacc_ref, acc_ref):
    b = pl.program_id(0)
    q = pl.program_id(1)
    x = bf_ref[0, 0]
    h_t = lax.dot_general(w1_ref[...], x, (((0,), (1,)), ((), ())),
                          preferred_element_type=jnp.float32) + b1_ref[...]
    h_t = jax.nn.gelu(h_t)
    lr_t = lax.dot_general(w2_ref[...], h_t, (((0,), (0,)), ((), ())),
                           preferred_element_type=jnp.float32) + b2_ref[...]
    lr_ref[0, 0] = lr_t
    mask = tokpos_ref[0] < lengths_ref[b]

    @pl.when((b == 0) & (q == 0))
    def _():
        acc_ref[0] = 0.0
        acc_ref[1] = 0.0

    acc_ref[0] += jnp.sum(jnp.where(mask, jnp.exp(lr_t), 0.0))
    acc_ref[1] += jnp.sum(mask.astype(jnp.float32))

    @pl.when((b == B - 1) & (q == QM - 1))
    def _():
        eacc_ref[0, 0] = acc_ref[0]
        eacc_ref[0, 1] = acc_ref[1]


_tc_mlp = pl.pallas_call(
    _tc_mlp_body,
    grid=(B, QM),
    in_specs=[
        pl.BlockSpec(memory_space=pltpu.SMEM),
        pl.BlockSpec((1, 1, 128), lambda b, q: (q, 0, 0)),
        pl.BlockSpec((1, 1, 128, H), lambda b, q: (b, q, 0, 0)),
        pl.BlockSpec((H, H), lambda b, q: (0, 0)),
        pl.BlockSpec((H, 1), lambda b, q: (0, 0)),
        pl.BlockSpec((H, C), lambda b, q: (0, 0)),
        pl.BlockSpec((C, 1), lambda b, q: (0, 0)),
    ],
    out_specs=[
        pl.BlockSpec((1, 1, C, 128), lambda b, q: (b, q, 0, 0)),
        pl.BlockSpec(memory_space=pltpu.SMEM),
    ],
    out_shape=[
        jax.ShapeDtypeStruct((B, QM, C, 128), jnp.float32),
        jax.ShapeDtypeStruct((1, 2), jnp.float32),
    ],
    scratch_shapes=[pltpu.SMEM((2,), jnp.float32)],
)


def _tc_loss_body(lengths_ref, tokpos_ref, eacc_ref, lr_ref, tgt_ref,
                  out_ref, acc_ref):
    b = pl.program_id(0)
    q = pl.program_id(1)
    lr_t = lr_ref[0, 0]
    tgt_t = tgt_ref[0, 0].astype(jnp.float32)
    mask = tokpos_ref[0] < lengths_ref[b]
    prod = jnp.where(mask, tgt_t * lr_t, 0.0)

    @pl.when((b == 0) & (q == 0))
    def _():
        acc_ref[0] = 0.0

    acc_ref[0] += jnp.sum(prod)

    @pl.when((b == B - 1) & (q == QM - 1))
    def _():
        denom = jnp.maximum(eacc_ref[0, 1] * C, 1.0)
        out_ref[0, 0] = (eacc_ref[0, 0] - acc_ref[0]) / denom


_tc_loss = pl.pallas_call(
    _tc_loss_body,
    grid=(B, QM),
    in_specs=[
        pl.BlockSpec(memory_space=pltpu.SMEM),
        pl.BlockSpec((1, 1, 128), lambda b, q: (q, 0, 0)),
        pl.BlockSpec(memory_space=pltpu.SMEM),
        pl.BlockSpec((1, 1, C, 128), lambda b, q: (b, q, 0, 0)),
        pl.BlockSpec((1, 1, C, 128), lambda b, q: (b, q, 0, 0)),
    ],
    out_specs=pl.BlockSpec(memory_space=pltpu.SMEM),
    out_shape=jax.ShapeDtypeStruct((1, 1), jnp.float32),
    scratch_shapes=[pltpu.SMEM((1,), jnp.float32)],
)


def kernel(backbone_features, spikes, shuffle, lengths, encoder_frac, W1, b1, W2, b2):
    del encoder_frac
    spikes_v = (spikes.reshape(B, T // 128, 128, C // 8, 8)
                .transpose(0, 3, 1, 4, 2))
    tgt_q = _sc_gather(shuffle, spikes_v)
    tokpos = shuffle[ENC:].reshape(QM, 1, 128)
    bf_q = backbone_features.reshape(B, QM, 128, H)
    lr_q, eacc = _tc_mlp(lengths, tokpos, bf_q, W1, b1.reshape(H, 1),
                         W2, b2.reshape(C, 1))
    out = _tc_loss(lengths, tokpos, eacc, lr_q, tgt_q)
    return out[0, 0]


# speedup vs baseline: 2.7590x; 2.7590x over previous
import functools

import jax
import jax.numpy as jnp
from jax import lax
from jax.experimental import pallas as pl
from jax.experimental.pallas import tpu as pltpu
from jax.experimental.pallas import tpu_sc as plsc

B, T, H, C = 8, 2048, 128, 32
ENC = 1024
TM = T - ENC
QM = TM // 128

NC, NS = 2, 16
NW = NC * NS
ROWS_PER_W = (B * C) // NW
QT = T // 128


def _sc_gather_body(shuffle_hbm, spikes_hbm, out_hbm, idx_v, plane_v, res_v, sem):
    wid = lax.axis_index("s") * NC + lax.axis_index("c")
    r0 = wid * ROWS_PER_W
    b = r0 // C
    c0 = r0 % C
    pltpu.sync_copy(shuffle_hbm.at[pl.ds(ENC, TM)], idx_v)
    cps = [pltpu.async_copy(
        spikes_hbm.at[b].at[c0 // 8].at[:, cl],
        plane_v.at[cl], sem) for cl in range(8)]
    for cp in cps:
        cp.wait()
    for g in range(TM // 16):
        t16 = idx_v[pl.ds(g * 16, 16)]
        q16 = lax.shift_right_logical(t16, 7)
        l16 = lax.bitwise_and(t16, 127)
        q_out = g // 8
        lanes = pl.ds((g % 8) * 16, 16)
        for cl in range(8):
            v16 = plsc.load_gather(plane_v, [jnp.full((16,), cl, jnp.int32), q16, l16])
            res_v[(cl, q_out, lanes)] = v16
    pltpu.sync_copy(res_v, out_hbm.at[b].at[pl.ds(c0, 8)])


_sc_gather = functools.partial(
    pl.kernel,
    mesh=plsc.VectorSubcoreMesh(core_axis_name="c", subcore_axis_name="s"),
    out_type=jax.ShapeDtypeStruct((B, C, QM, 128), jnp.int32),
    scratch_types=[
        pltpu.VMEM((TM,), jnp.int32),
        pltpu.VMEM((8, QT, 128), jnp.int32),
        pltpu.VMEM((8, QM, 128), jnp.int32),
        pltpu.SemaphoreType.DMA,
    ],
    compiler_params=pltpu.CompilerParams(use_tc_tiling_on_sc=False,
                                         needs_layout_passes=False),
)(_sc_gather_body)


def _tc_mlp_body(lengths_ref, tokpos_ref, bf_ref, w1_ref, b1_ref, w2_ref, b2_ref,
                 lr_ref, eacc_ref, acc_ref):
    b = pl.program_id(0)
    x = bf_ref[0]
    h_t = lax.dot_general(w1_ref[...], x, (((0,), (2,)), ((), ())),
                          preferred_element_type=jnp.float32) + b1_ref[...]
    h_t = jax.nn.gelu(h_t)
    lr_t = lax.dot_general(w2_ref[...], h_t, (((0,), (0,)), ((), ())),
                           preferred_element_type=jnp.float32) + b2_ref[...]
    lr_ref[0] = lr_t
    mask = tokpos_ref[...] < lengths_ref[b]

    @pl.when(b == 0)
    def _():
        acc_ref[0] = 0.0
        acc_ref[1] = 0.0

    acc_ref[0] += jnp.sum(jnp.where(mask, jnp.exp(lr_t), 0.0))
    acc_ref[1] += jnp.sum(mask.astype(jnp.float32))

    @pl.when(b == B - 1)
    def _():
        eacc_ref[0, 0] = acc_ref[0]
        eacc_ref[0, 1] = acc_ref[1]


_tc_mlp = pl.pallas_call(
    _tc_mlp_body,
    grid=(B,),
    in_specs=[
        pl.BlockSpec(memory_space=pltpu.SMEM),
        pl.BlockSpec((1, QM, 128), lambda b: (0, 0, 0)),
        pl.BlockSpec((1, QM, 128, H), lambda b: (b, 0, 0, 0)),
        pl.BlockSpec((H, H), lambda b: (0, 0)),
        pl.BlockSpec((H, 1, 1), lambda b: (0, 0, 0)),
        pl.BlockSpec((H, C), lambda b: (0, 0)),
        pl.BlockSpec((C, 1, 1), lambda b: (0, 0, 0)),
    ],
    out_specs=[
        pl.BlockSpec((1, C, QM, 128), lambda b: (b, 0, 0, 0)),
        pl.BlockSpec(memory_space=pltpu.SMEM),
    ],
    out_shape=[
        jax.ShapeDtypeStruct((B, C, QM, 128), jnp.float32),
        jax.ShapeDtypeStruct((1, 2), jnp.float32),
    ],
    scratch_shapes=[pltpu.SMEM((2,), jnp.float32)],
)


def _tc_loss_body(lengths_ref, tokpos_ref, eacc_ref, lr_ref, tgt_ref,
                  out_ref, acc_ref):
    b = pl.program_id(0)
    lr_t = lr_ref[0]
    tgt_t = tgt_ref[0].astype(jnp.float32)
    mask = tokpos_ref[...] < lengths_ref[b]
    prod = jnp.where(mask, tgt_t * lr_t, 0.0)

    @pl.when(b == 0)
    def _():
        acc_ref[0] = 0.0

    acc_ref[0] += jnp.sum(prod)

    @pl.when(b == B - 1)
    def _():
        denom = jnp.maximum(eacc_ref[0, 1] * C, 1.0)
        out_ref[0, 0] = (eacc_ref[0, 0] - acc_ref[0]) / denom


_tc_loss = pl.pallas_call(
    _tc_loss_body,
    grid=(B,),
    in_specs=[
        pl.BlockSpec(memory_space=pltpu.SMEM),
        pl.BlockSpec((1, QM, 128), lambda b: (0, 0, 0)),
        pl.BlockSpec(memory_space=pltpu.SMEM),
        pl.BlockSpec((1, C, QM, 128), lambda b: (b, 0, 0, 0)),
        pl.BlockSpec((1, C, QM, 128), lambda b: (b, 0, 0, 0)),
    ],
    out_specs=pl.BlockSpec(memory_space=pltpu.SMEM),
    out_shape=jax.ShapeDtypeStruct((1, 1), jnp.float32),
    scratch_shapes=[pltpu.SMEM((1,), jnp.float32)],
)


def kernel(backbone_features, spikes, shuffle, lengths, encoder_frac, W1, b1, W2, b2):
    del encoder_frac
    spikes_v = (spikes.reshape(B, T // 128, 128, C // 8, 8)
                .transpose(0, 3, 1, 4, 2))
    tgt_q = _sc_gather(shuffle, spikes_v)
    tokpos = shuffle[ENC:].reshape(1, QM, 128)
    bf_q = backbone_features.reshape(B, QM, 128, H)
    lr_q, eacc = _tc_mlp(lengths, tokpos, bf_q, W1, b1.reshape(H, 1, 1),
                         W2, b2.reshape(C, 1, 1))
    out = _tc_loss(lengths, tokpos, eacc, lr_q, tgt_q)
    return out[0, 0]


# speedup vs baseline: 2.9173x; 1.0573x over previous
import functools

import jax
import jax.numpy as jnp
from jax import lax
from jax.experimental import pallas as pl
from jax.experimental.pallas import tpu as pltpu
from jax.experimental.pallas import tpu_sc as plsc

B, T, H, C = 8, 2048, 128, 32
ENC = 1024
TM = T - ENC
QM = TM // 128

NC, NS = 2, 16
NW = NC * NS
ROWS_PER_W = (B * C) // NW
QT = T // 128


def _sc_gather_body(shuffle_hbm, spikes_hbm, out_hbm, idx_v, plane_v, res_v, sem):
    wid = lax.axis_index("s") * NC + lax.axis_index("c")
    r0 = wid * ROWS_PER_W
    b = r0 // C
    c0 = r0 % C
    pltpu.sync_copy(shuffle_hbm.at[pl.ds(ENC, TM)], idx_v)
    cps = [pltpu.async_copy(
        spikes_hbm.at[b].at[c0 // 8].at[:, cl],
        plane_v.at[cl], sem) for cl in range(8)]
    for cp in cps:
        cp.wait()
    for g in range(TM // 16):
        t16 = idx_v[pl.ds(g * 16, 16)]
        q16 = lax.shift_right_logical(t16, 7)
        l16 = lax.bitwise_and(t16, 127)
        q_out = g // 8
        lanes = pl.ds((g % 8) * 16, 16)
        for cl in range(8):
            v16 = plsc.load_gather(plane_v, [jnp.full((16,), cl, jnp.int32), q16, l16])
            res_v[(cl, q_out, lanes)] = v16
    pltpu.sync_copy(res_v, out_hbm.at[b].at[pl.ds(c0, 8)])


_sc_gather = functools.partial(
    pl.kernel,
    mesh=plsc.VectorSubcoreMesh(core_axis_name="c", subcore_axis_name="s"),
    out_type=jax.ShapeDtypeStruct((B, C, QM, 128), jnp.int32),
    scratch_types=[
        pltpu.VMEM((TM,), jnp.int32),
        pltpu.VMEM((8, QT, 128), jnp.int32),
        pltpu.VMEM((8, QM, 128), jnp.int32),
        pltpu.SemaphoreType.DMA,
    ],
    compiler_params=pltpu.CompilerParams(use_tc_tiling_on_sc=False,
                                         needs_layout_passes=False),
)(_sc_gather_body)


def _tc_mlp_body(lengths_ref, tokpos_ref, bf_ref, w1_ref, b1_ref, w2_ref, b2_ref,
                 lr_ref, eacc_ref, acc_ref):
    b = pl.program_id(0)
    x = bf_ref[0]
    h_t = lax.dot_general(w1_ref[...], x, (((0,), (2,)), ((), ())),
                          preferred_element_type=jnp.float32) + b1_ref[...]
    h_t = jax.nn.gelu(h_t)
    lr_t = lax.dot_general(w2_ref[...], h_t, (((0,), (0,)), ((), ())),
                           preferred_element_type=jnp.float32) + b2_ref[...]
    mask = tokpos_ref[...] < lengths_ref[b]
    lr_ref[0] = jnp.where(mask, lr_t, 0.0)

    @pl.when(b == 0)
    def _():
        acc_ref[0] = 0.0
        acc_ref[1] = 0.0

    acc_ref[0] += jnp.sum(jnp.where(mask, jnp.exp(lr_t), 0.0))
    acc_ref[1] += jnp.sum(mask.astype(jnp.float32))

    @pl.when(b == B - 1)
    def _():
        eacc_ref[0, 0] = acc_ref[0]
        eacc_ref[0, 1] = acc_ref[1]


_tc_mlp = pl.pallas_call(
    _tc_mlp_body,
    grid=(B,),
    in_specs=[
        pl.BlockSpec(memory_space=pltpu.SMEM),
        pl.BlockSpec((1, QM, 128), lambda b: (0, 0, 0)),
        pl.BlockSpec((1, QM, 128, H), lambda b: (b, 0, 0, 0)),
        pl.BlockSpec((H, H), lambda b: (0, 0)),
        pl.BlockSpec((H, 1, 1), lambda b: (0, 0, 0)),
        pl.BlockSpec((H, C), lambda b: (0, 0)),
        pl.BlockSpec((C, 1, 1), lambda b: (0, 0, 0)),
    ],
    out_specs=[
        pl.BlockSpec((1, C, QM, 128), lambda b: (b, 0, 0, 0)),
        pl.BlockSpec(memory_space=pltpu.SMEM),
    ],
    out_shape=[
        jax.ShapeDtypeStruct((B, C, QM, 128), jnp.float32),
        jax.ShapeDtypeStruct((1, 2), jnp.float32),
    ],
    scratch_shapes=[pltpu.SMEM((2,), jnp.float32)],
)


def _tc_loss_body(eacc_ref, lr_ref, tgt_ref, out_ref, acc_ref):
    g = pl.program_id(0)
    prod = tgt_ref[...].astype(jnp.float32) * lr_ref[...]

    @pl.when(g == 0)
    def _():
        acc_ref[0] = 0.0

    acc_ref[0] += jnp.sum(prod)

    @pl.when(g == B // 2 - 1)
    def _():
        denom = jnp.maximum(eacc_ref[0, 1] * C, 1.0)
        out_ref[0, 0] = (eacc_ref[0, 0] - acc_ref[0]) / denom


_tc_loss = pl.pallas_call(
    _tc_loss_body,
    grid=(B // 2,),
    in_specs=[
        pl.BlockSpec(memory_space=pltpu.SMEM),
        pl.BlockSpec((2, C, QM, 128), lambda g: (g, 0, 0, 0)),
        pl.BlockSpec((2, C, QM, 128), lambda g: (g, 0, 0, 0)),
    ],
    out_specs=pl.BlockSpec(memory_space=pltpu.SMEM),
    out_shape=jax.ShapeDtypeStruct((1, 1), jnp.float32),
    scratch_shapes=[pltpu.SMEM((1,), jnp.float32)],
)


def kernel(backbone_features, spikes, shuffle, lengths, encoder_frac, W1, b1, W2, b2):
    del encoder_frac
    spikes_v = (spikes.reshape(B, T // 128, 128, C // 8, 8)
                .transpose(0, 3, 1, 4, 2))
    tgt_q = _sc_gather(shuffle, spikes_v)
    tokpos = shuffle[ENC:].reshape(1, QM, 128)
    bf_q = backbone_features.reshape(B, QM, 128, H)
    lr_q, eacc = _tc_mlp(lengths, tokpos, bf_q, W1, b1.reshape(H, 1, 1),
                         W2, b2.reshape(C, 1, 1))
    out = _tc_loss(eacc, lr_q, tgt_q)
    return out[0, 0]


# speedup vs baseline: 3.0031x; 1.0294x over previous
import functools

import jax
import jax.numpy as jnp
from jax import lax
from jax.experimental import pallas as pl
from jax.experimental.pallas import tpu as pltpu
from jax.experimental.pallas import tpu_sc as plsc

B, T, H, C = 8, 2048, 128, 32
ENC = 1024
TM = T - ENC
QM = TM // 128

NC, NS = 2, 16
NW = NC * NS
ROWS_PER_W = (B * C) // NW
QT = T // 128


def _sc_gather_body(shuffle_hbm, spikes_hbm, out_hbm, idx_v, plane_v, res_v, sem):
    wid = lax.axis_index("s") * NC + lax.axis_index("c")
    r0 = wid * ROWS_PER_W
    b = r0 // C
    c0 = r0 % C
    pltpu.sync_copy(shuffle_hbm.at[pl.ds(ENC, TM)], idx_v)
    cps = [pltpu.async_copy(
        spikes_hbm.at[b].at[c0 // 8].at[:, cl],
        plane_v.at[cl], sem) for cl in range(8)]
    for cp in cps:
        cp.wait()
    for g in range(TM // 16):
        t16 = idx_v[pl.ds(g * 16, 16)]
        q16 = lax.shift_right_logical(t16, 7)
        l16 = lax.bitwise_and(t16, 127)
        q_out = g // 8
        lanes = pl.ds((g % 8) * 16, 16)
        for cl in range(8):
            v16 = plsc.load_gather(plane_v, [jnp.full((16,), cl, jnp.int32), q16, l16])
            res_v[(cl, q_out, lanes)] = v16
    pltpu.sync_copy(res_v, out_hbm.at[b].at[pl.ds(c0, 8)])


_sc_gather = functools.partial(
    pl.kernel,
    mesh=plsc.VectorSubcoreMesh(core_axis_name="c", subcore_axis_name="s"),
    out_type=jax.ShapeDtypeStruct((B, C, QM, 128), jnp.int32),
    scratch_types=[
        pltpu.VMEM((TM,), jnp.int32),
        pltpu.VMEM((8, QT, 128), jnp.int32),
        pltpu.VMEM((8, QM, 128), jnp.int32),
        pltpu.SemaphoreType.DMA,
    ],
    compiler_params=pltpu.CompilerParams(use_tc_tiling_on_sc=False,
                                         needs_layout_passes=False),
)(_sc_gather_body)


def _tc_mlp_body(lengths_ref, tokpos_ref, bf_ref, w1_ref, b1_ref, w2_ref, b2_ref,
                 lr_ref, eacc_ref, acc_ref):
    b = pl.program_id(0)
    x = bf_ref[0]
    h_t = lax.dot_general(w1_ref[...], x, (((0,), (2,)), ((), ())),
                          preferred_element_type=jnp.float32) + b1_ref[...]
    h_t = jax.nn.gelu(h_t)
    lr_t = lax.dot_general(w2_ref[...], h_t, (((1,), (0,)), ((), ())),
                           preferred_element_type=jnp.float32) + b2_ref[...]
    mask = tokpos_ref[...] < lengths_ref[b]
    lr_ref[0] = jnp.where(mask, lr_t, 0.0)

    @pl.when(b == 0)
    def _():
        acc_ref[0] = 0.0
        acc_ref[1] = 0.0

    acc_ref[0] += jnp.sum(jnp.where(mask, jnp.exp(lr_t), 0.0))
    acc_ref[1] += jnp.sum(mask.astype(jnp.float32))

    @pl.when(b == B - 1)
    def _():
        eacc_ref[0, 0] = acc_ref[0]
        eacc_ref[0, 1] = acc_ref[1]


_tc_mlp = pl.pallas_call(
    _tc_mlp_body,
    grid=(B,),
    in_specs=[
        pl.BlockSpec(memory_space=pltpu.SMEM),
        pl.BlockSpec((1, QM, 128), lambda b: (0, 1, 0)),
        pl.BlockSpec((1, QM, 128, H), lambda b: (b, 0, 0, 0)),
        pl.BlockSpec((H, H), lambda b: (0, 0)),
        pl.BlockSpec((H, 1, 1), lambda b: (0, 0, 0)),
        pl.BlockSpec((C, H), lambda b: (0, 0)),
        pl.BlockSpec((C, 1, 1), lambda b: (0, 0, 0)),
    ],
    out_specs=[
        pl.BlockSpec((1, C, QM, 128), lambda b: (b, 0, 0, 0)),
        pl.BlockSpec(memory_space=pltpu.SMEM),
    ],
    out_shape=[
        jax.ShapeDtypeStruct((B, C, QM, 128), jnp.float32),
        jax.ShapeDtypeStruct((1, 2), jnp.float32),
    ],
    scratch_shapes=[pltpu.SMEM((2,), jnp.float32)],
)


def _tc_loss_body(eacc_ref, lr_ref, tgt_ref, out_ref, acc_ref):
    g = pl.program_id(0)
    prod = tgt_ref[...].astype(jnp.float32) * lr_ref[...]

    @pl.when(g == 0)
    def _():
        acc_ref[0] = 0.0

    acc_ref[0] += jnp.sum(prod)

    @pl.when(g == B // 2 - 1)
    def _():
        denom = jnp.maximum(eacc_ref[0, 1] * C, 1.0)
        out_ref[0, 0] = (eacc_ref[0, 0] - acc_ref[0]) / denom


_tc_loss = pl.pallas_call(
    _tc_loss_body,
    grid=(B // 2,),
    in_specs=[
        pl.BlockSpec(memory_space=pltpu.SMEM),
        pl.BlockSpec((2, C, QM, 128), lambda g: (g, 0, 0, 0)),
        pl.BlockSpec((2, C, QM, 128), lambda g: (g, 0, 0, 0)),
    ],
    out_specs=pl.BlockSpec(memory_space=pltpu.SMEM),
    out_shape=jax.ShapeDtypeStruct((1, 1), jnp.float32),
    scratch_shapes=[pltpu.SMEM((1,), jnp.float32)],
)


def kernel(backbone_features, spikes, shuffle, lengths, encoder_frac, W1, b1, W2, b2):
    del encoder_frac
    spikes_v = (spikes.reshape(B, T // 128, 128, C // 8, 8)
                .transpose(0, 3, 1, 4, 2))
    tgt_q = _sc_gather(shuffle, spikes_v)
    shuf_v = shuffle.reshape(1, T // 128, 128)
    bf_q = backbone_features.reshape(B, QM, 128, H)
    lr_q, eacc = _tc_mlp(lengths, shuf_v, bf_q, W1, b1.reshape(H, 1, 1),
                         W2.T, b2.reshape(C, 1, 1))
    out = _tc_loss(eacc, lr_q, tgt_q)
    return out[0, 0]


# speedup vs baseline: 3.1058x; 1.0342x over previous
import functools

import jax
import jax.numpy as jnp
from jax import lax
from jax.experimental import pallas as pl
from jax.experimental.pallas import tpu as pltpu
from jax.experimental.pallas import tpu_sc as plsc

B, T, H, C = 8, 2048, 128, 32
ENC = 1024
TM = T - ENC
QM = TM // 128

NC, NS = 2, 16
NW = NC * NS
ROWS_PER_W = (B * C) // NW
QT = T // 128


def _sc_gather_body(shuffle_hbm, spikes_hbm, out_hbm, idx_v, plane_v, res_v, sem):
    wid = lax.axis_index("s") * NC + lax.axis_index("c")
    r0 = wid * ROWS_PER_W
    b = r0 // C
    c0 = r0 % C
    pltpu.sync_copy(shuffle_hbm.at[pl.ds(ENC, TM)], idx_v)
    cps = [pltpu.async_copy(
        spikes_hbm.at[b].at[c0 // 8].at[:, cl],
        plane_v.at[cl], sem) for cl in range(8)]
    for cp in cps:
        cp.wait()
    for g in range(TM // 16):
        t16 = idx_v[pl.ds(g * 16, 16)]
        q16 = lax.shift_right_logical(t16, 7)
        l16 = lax.bitwise_and(t16, 127)
        q_out = g // 8
        lanes = pl.ds((g % 8) * 16, 16)
        for cl in range(8):
            v16 = plsc.load_gather(plane_v, [jnp.full((16,), cl, jnp.int32), q16, l16])
            res_v[(cl, q_out, lanes)] = v16
    pltpu.sync_copy(res_v, out_hbm.at[b].at[pl.ds(c0, 8)])


_sc_gather = functools.partial(
    pl.kernel,
    mesh=plsc.VectorSubcoreMesh(core_axis_name="c", subcore_axis_name="s"),
    out_type=jax.ShapeDtypeStruct((B, C, QM, 128), jnp.int32),
    scratch_types=[
        pltpu.VMEM((TM,), jnp.int32),
        pltpu.VMEM((8, QT, 128), jnp.int32),
        pltpu.VMEM((8, QM, 128), jnp.int32),
        pltpu.SemaphoreType.DMA,
    ],
    compiler_params=pltpu.CompilerParams(use_tc_tiling_on_sc=False,
                                         needs_layout_passes=False),
)(_sc_gather_body)


def _tc_mlp_body(lengths_ref, tokpos_ref, bf_ref, w1_ref, b1_ref, w2_ref, b2_ref,
                 lr_ref, eacc_ref, acc_ref):
    b = pl.program_id(0)
    x = bf_ref[0]
    h_t = lax.dot_general(w1_ref[...], x, (((0,), (2,)), ((), ())),
                          preferred_element_type=jnp.float32) + b1_ref[...]
    h_t = jax.nn.gelu(h_t)
    lr_t = lax.dot_general(w2_ref[...], h_t, (((1,), (0,)), ((), ())),
                           preferred_element_type=jnp.float32) + b2_ref[...]
    mask = tokpos_ref[...] < lengths_ref[b]
    lr_ref[0] = jnp.where(mask, lr_t, 0.0)

    @pl.when(b == 0)
    def _():
        acc_ref[0] = 0.0
        acc_ref[1] = 0.0

    acc_ref[0] += jnp.sum(jnp.where(mask, jnp.exp(lr_t), 0.0))
    acc_ref[1] += jnp.sum(mask.astype(jnp.float32))

    @pl.when(b == B - 1)
    def _():
        eacc_ref[0, 0] = acc_ref[0]
        eacc_ref[0, 1] = acc_ref[1]


_tc_mlp = pl.pallas_call(
    _tc_mlp_body,
    grid=(B,),
    in_specs=[
        pl.BlockSpec(memory_space=pltpu.SMEM),
        pl.BlockSpec((1, QM, 128), lambda b: (0, 1, 0)),
        pl.BlockSpec((1, QM, 128, H), lambda b: (b, 0, 0, 0)),
        pl.BlockSpec((H, H), lambda b: (0, 0)),
        pl.BlockSpec((H, 1, 1), lambda b: (0, 0, 0)),
        pl.BlockSpec((C, H), lambda b: (0, 0)),
        pl.BlockSpec((C, 1, 1), lambda b: (0, 0, 0)),
    ],
    out_specs=[
        pl.BlockSpec((1, C, QM, 128), lambda b: (b, 0, 0, 0)),
        pl.BlockSpec(memory_space=pltpu.SMEM),
    ],
    out_shape=[
        jax.ShapeDtypeStruct((B, C, QM, 128), jnp.float32),
        jax.ShapeDtypeStruct((1, 2), jnp.float32),
    ],
    scratch_shapes=[pltpu.SMEM((2,), jnp.float32)],
)


def _tc_loss_body(eacc_ref, lr_ref, tgt_ref, out_ref, acc_ref):
    g = pl.program_id(0)
    prod = tgt_ref[...].astype(jnp.float32) * lr_ref[...]

    @pl.when(g == 0)
    def _():
        acc_ref[0] = 0.0

    acc_ref[0] += jnp.sum(prod)

    @pl.when(g == B // 4 - 1)
    def _():
        denom = jnp.maximum(eacc_ref[0, 1] * C, 1.0)
        out_ref[0, 0] = (eacc_ref[0, 0] - acc_ref[0]) / denom


_tc_loss = pl.pallas_call(
    _tc_loss_body,
    grid=(B // 4,),
    in_specs=[
        pl.BlockSpec(memory_space=pltpu.SMEM),
        pl.BlockSpec((4, C, QM, 128), lambda g: (g, 0, 0, 0)),
        pl.BlockSpec((4, C, QM, 128), lambda g: (g, 0, 0, 0)),
    ],
    out_specs=pl.BlockSpec(memory_space=pltpu.SMEM),
    out_shape=jax.ShapeDtypeStruct((1, 1), jnp.float32),
    scratch_shapes=[pltpu.SMEM((1,), jnp.float32)],
)


def kernel(backbone_features, spikes, shuffle, lengths, encoder_frac, W1, b1, W2, b2):
    del encoder_frac
    spikes_v = (spikes.reshape(B, T // 128, 128, C // 8, 8)
                .transpose(0, 3, 1, 4, 2))
    tgt_q = _sc_gather(shuffle, spikes_v)
    shuf_v = shuffle.reshape(1, T // 128, 128)
    bf_q = backbone_features.reshape(B, QM, 128, H)
    lr_q, eacc = _tc_mlp(lengths, shuf_v, bf_q, W1, b1.reshape(H, 1, 1),
                         W2.T, b2.reshape(C, 1, 1))
    out = _tc_loss(eacc, lr_q, tgt_q)
    return out[0, 0]
